# baseline (speedup 1.0000x reference)
import functools

import jax
import jax.numpy as jnp
from jax import lax
from jax.experimental import pallas as pl
from jax.experimental.pallas import tpu as pltpu
from jax.experimental.pallas import tpu_sc as plsc

B, S, C = 2, 2048, 768
E, K, FF = 8, 2, 2048
N = B * C
SP = S // 2
BLK = 128
NB = 32
NP = NB * BLK
TT = 512
NT = S // TT
FT = 512

_NC, _NS = 2, 16
_NW = _NC * _NS
GCH = 32
CCH = 8


def _pack_rows(x):
    u = lax.bitcast_convert_type(x, jnp.uint32)
    r = (u + jnp.uint32(0x7FFF) + ((u >> 16) & jnp.uint32(1))) >> 16
    k = x.shape[-1] // 2
    p = r[..., :k] | (r[..., k:] << 16)
    return lax.bitcast_convert_type(p, jnp.int32)


def _unpack_rows(p):
    u = lax.bitcast_convert_type(p, jnp.uint32)
    lo = lax.bitcast_convert_type(u << 16, jnp.float32)
    hi = lax.bitcast_convert_type(u & jnp.uint32(0xFFFF0000), jnp.float32)
    return jnp.concatenate([lo, hi], axis=-1)


def _gating_body(xt_ref, wg_ref, gate_ref, w0_ref, w1_ref, e0_ref, e1_ref,
                 xtpk_ref):
    xtb = xt_ref[0]
    xtpk_ref[0] = _pack_rows(xtb)
    gl = lax.dot_general(xtb, wg_ref[...], (((1,), (1,)), ((), ())),
                         preferred_element_type=jnp.float32)
    m = jnp.max(gl, axis=1, keepdims=True)
    ex = jnp.exp(gl - m)
    gate = ex / jnp.sum(ex, axis=1, keepdims=True)
    gate_ref[0] = gate
    iota = lax.broadcasted_iota(jnp.int32, (C, E), 1)
    m0 = jnp.max(gate, axis=1, keepdims=True)
    i0 = jnp.min(jnp.where(gate == m0, iota, E), axis=1, keepdims=True)
    g2 = jnp.where(iota == i0, -jnp.inf, gate)
    m1 = jnp.max(g2, axis=1, keepdims=True)
    i1 = jnp.min(jnp.where(g2 == m1, iota, E), axis=1, keepdims=True)
    w0_ref[0] = m0
    w1_ref[0] = m1
    e0_ref[0] = i0
    e1_ref[0] = i1


def _gating(xt, Wg):
    out_shapes = (
        jax.ShapeDtypeStruct((B, C, E), jnp.float32),
        jax.ShapeDtypeStruct((B, C, 1), jnp.float32),
        jax.ShapeDtypeStruct((B, C, 1), jnp.float32),
        jax.ShapeDtypeStruct((B, C, 1), jnp.int32),
        jax.ShapeDtypeStruct((B, C, 1), jnp.int32),
        jax.ShapeDtypeStruct((B, C, SP), jnp.int32),
    )
    return pl.pallas_call(
        _gating_body,
        grid=(B,),
        in_specs=[
            pl.BlockSpec((1, C, S), lambda b: (b, 0, 0)),
            pl.BlockSpec((E, S), lambda b: (0, 0)),
        ],
        out_specs=(
            pl.BlockSpec((1, C, E), lambda b: (b, 0, 0)),
            pl.BlockSpec((1, C, 1), lambda b: (b, 0, 0)),
            pl.BlockSpec((1, C, 1), lambda b: (b, 0, 0)),
            pl.BlockSpec((1, C, 1), lambda b: (b, 0, 0)),
            pl.BlockSpec((1, C, 1), lambda b: (b, 0, 0)),
            pl.BlockSpec((1, C, SP), lambda b: (b, 0, 0)),
        ),
        out_shape=out_shapes,
    )(xt, Wg)


def _routing_meta(e0, e1, w0, w1):
    eP = jnp.concatenate([e0, e1])
    wP = jnp.concatenate([w0, w1])
    tokP = jnp.concatenate([jnp.arange(N, dtype=jnp.int32)] * 2)
    oh = (eP[:, None] == jnp.arange(E, dtype=jnp.int32)[None, :]).astype(jnp.int32)
    csum = jnp.cumsum(oh, axis=0)
    cnt = csum[-1]
    rank = jnp.take_along_axis(csum, eP[:, None], axis=1)[:, 0] - 1
    blocks_e = (cnt + BLK - 1) // BLK
    bcum = jnp.cumsum(blocks_e)
    bstart = bcum - blocks_e
    pos = bstart[eP] * BLK + rank
    idxg = jnp.zeros((NP,), jnp.int32).at[pos].set(tokP)
    rw = jnp.zeros((NP,), jnp.float32).at[pos].set(wP)
    bexp = jnp.minimum(
        jnp.searchsorted(bcum, jnp.arange(NB, dtype=jnp.int32), side="right"),
        E - 1).astype(jnp.int32)
    nb = bcum[-1:].astype(jnp.int32)
    return idxg, rw, bexp, nb, pos[:N], pos[N:]


def _sc_gather(tab, idx_h):
    mesh = plsc.VectorSubcoreMesh(core_axis_name="c", subcore_axis_name="s")
    dt = tab.dtype
    D = tab.shape[1]
    nslots = idx_h.shape[0]
    rpw = nslots // _NW
    nch = rpw // GCH

    @functools.partial(
        pl.kernel,
        out_type=jax.ShapeDtypeStruct((nslots, D), dt),
        mesh=mesh,
        scratch_types=[
            pltpu.VMEM((rpw,), jnp.int32),
            pltpu.VMEM((GCH, D), dt),
            pltpu.VMEM((GCH, D), dt),
            pltpu.VMEM((GCH, D), dt),
        ] + [pltpu.SemaphoreType.DMA] * 6,
    )
    def k(tab_hbm, idx_hbm, out_hbm, idx_all, r0, r1, r2,
          sg0, sg1, sg2, ss0, ss1, ss2):
        wid = lax.axis_index("s") * _NC + lax.axis_index("c")
        base = wid * rpw
        rows = [r0, r1, r2]
        semg = [sg0, sg1, sg2]
        sems = [ss0, ss1, ss2]
        g = [None, None, None]
        s = [None, None, None]
        pltpu.sync_copy(idx_hbm.at[pl.ds(base, rpw)], idx_all)

        def gstart(ci, b):
            g[b] = pltpu.async_copy(
                tab_hbm.at[idx_all.at[pl.ds(ci * GCH, GCH)]], rows[b], semg[b])

        for b in range(min(3, nch)):
            gstart(b, b)
        for ci in range(nch):
            b = ci % 3
            g[b].wait()
            s[b] = pltpu.async_copy(
                rows[b], out_hbm.at[pl.ds(base + ci * GCH, GCH)], sems[b])
            if ci + 3 < nch:
                s[b].wait()
                gstart(ci + 3, b)
        for ci in range(max(0, nch - 3), nch):
            s[ci % 3].wait()

    return k(tab, idx_h)


def _mm_body(bexp_sm, nb_sm, xs_ref, we_ref, rw_ref, be_ref, out_ref):
    j = pl.program_id(0)

    @pl.when(j < nb_sm[0])
    def _():
        x = _unpack_rows(xs_ref[pl.ds(j * BLK, BLK), :])
        w = we_ref[0]
        acc = lax.dot_general(x, w, (((1,), (1,)), ((), ())),
                              preferred_element_type=jnp.float32)
        out_ref[...] = (acc + be_ref[0]) * rw_ref[...]


def _expert_mm(xs_pk, We, be, rw, bexp, nb):
    be3 = be.reshape(E, 1, S)
    rw2 = rw.reshape(NP, 1)
    grid_spec = pltpu.PrefetchScalarGridSpec(
        num_scalar_prefetch=2,
        grid=(NB,),
        in_specs=[
            pl.BlockSpec((NP, SP), lambda j, bexp, nb: (0, 0)),
            pl.BlockSpec((1, S, S), lambda j, bexp, nb: (bexp[j], 0, 0)),
            pl.BlockSpec((BLK, 1), lambda j, bexp, nb: (j, 0)),
            pl.BlockSpec((1, 1, S), lambda j, bexp, nb: (bexp[j], 0, 0)),
        ],
        out_specs=pl.BlockSpec((BLK, S), lambda j, bexp, nb: (j, 0)),
    )
    return pl.pallas_call(
        _mm_body,
        grid_spec=grid_spec,
        out_shape=jax.ShapeDtypeStruct((NP, S), jnp.float32),
        compiler_params=pltpu.CompilerParams(
            dimension_semantics=("arbitrary",)),
    )(bexp, nb, xs_pk, We, rw2, be3)


def _sc_combine(partial, xt_sub, posA, posB):
    mesh = plsc.VectorSubcoreMesh(core_axis_name="c", subcore_axis_name="s")
    ntok = posA.shape[0]
    tpw = ntok // _NW
    nchc = tpw // CCH

    @functools.partial(
        pl.kernel,
        out_type=jax.ShapeDtypeStruct((ntok, S), jnp.float32),
        mesh=mesh,
        scratch_types=[
            pltpu.VMEM((tpw,), jnp.int32),
            pltpu.VMEM((tpw,), jnp.int32),
            pltpu.VMEM((CCH, S), jnp.float32),
            pltpu.VMEM((CCH, S), jnp.float32),
            pltpu.VMEM((CCH, S), jnp.float32),
            pltpu.VMEM((CCH, S), jnp.float32),
            pltpu.VMEM((CCH, S), jnp.float32),
            pltpu.VMEM((CCH, S), jnp.float32),
        ] + [pltpu.SemaphoreType.DMA] * 8,
    )
    def k(p_hbm, xt_hbm, pa_hbm, pb_hbm, out_hbm,
          ia_all, ib_all, pva0, pva1, pvb0, pvb1, xtv0, xtv1,
          spa0, spa1, spb0, spb1, sx0, sx1, so0, so1):
        wid = lax.axis_index("s") * _NC + lax.axis_index("c")
        base = wid * tpw
        pva = [pva0, pva1]
        pvb = [pvb0, pvb1]
        xtv = [xtv0, xtv1]
        spa = [spa0, spa1]
        spb = [spb0, spb1]
        sx = [sx0, sx1]
        sso = [so0, so1]
        ga = [None, None]
        gb = [None, None]
        gx = [None, None]
        so = [None, None]
        pltpu.sync_copy(pa_hbm.at[pl.ds(base, tpw)], ia_all)
        pltpu.sync_copy(pb_hbm.at[pl.ds(base, tpw)], ib_all)

        def start(ci, b):
            off = base + ci * CCH
            ga[b] = pltpu.async_copy(
                p_hbm.at[ia_all.at[pl.ds(ci * CCH, CCH)]], pva[b], spa[b])
            gb[b] = pltpu.async_copy(
                p_hbm.at[ib_all.at[pl.ds(ci * CCH, CCH)]], pvb[b], spb[b])
            gx[b] = pltpu.async_copy(
                xt_hbm.at[pl.ds(off, CCH)], xtv[b], sx[b])

        start(0, 0)
        for ci in range(nchc):
            b = ci % 2
            if ci + 1 < nchc:
                nb_ = (ci + 1) % 2
                if so[nb_] is not None:
                    so[nb_].wait()
                    so[nb_] = None
                start(ci + 1, nb_)
            ga[b].wait()
            gb[b].wait()
            gx[b].wait()
            for ti in range(CCH):
                def colbody(cc, carry2, ti=ti, b=b):
                    cs = cc * 16
                    va = pva[b][ti, pl.ds(cs, 16)]
                    vb = pvb[b][ti, pl.ds(cs, 16)]
                    xv = xtv[b][ti, pl.ds(cs, 16)]
                    xtv[b][ti, pl.ds(cs, 16)] = jnp.maximum(va + vb, 0.0) + xv
                    return carry2
                lax.fori_loop(0, S // 16, colbody, 0, unroll=8)
            so[b] = pltpu.async_copy(
                xtv[b], out_hbm.at[pl.ds(base + ci * CCH, CCH)], sso[b])
        for b in range(2):
            if so[b] is not None:
                so[b].wait()

    return k(partial, xt_sub, posA, posB)


def _ffn_body(x2_ref, w1_ref, b1_ref, w2_ref, b2_ref, out_ref):
    xb = x2_ref[0]
    acc = jnp.zeros((C, TT), jnp.float32)
    for fi in range(FF // FT):
        h = lax.dot_general(w1_ref[pl.ds(fi * FT, FT), :], xb,
                            (((1,), (0,)), ((), ())),
                            preferred_element_type=jnp.float32)
        h = jnp.maximum(h + b1_ref[pl.ds(fi * FT, FT), :], 0.0)
        acc = acc + lax.dot_general(w2_ref[:, pl.ds(fi * FT, FT)], h,
                                    (((1,), (0,)), ((), ())),
                                    preferred_element_type=jnp.float32)
    out_ref[0] = jnp.transpose(acc + b2_ref[...] + xb)


def _ffn(x2, W1, b1, W2, b2):
    return pl.pallas_call(
        _ffn_body,
        grid=(B * NT,),
        in_specs=[
            pl.BlockSpec((1, C, TT), lambda i: (i // NT, 0, i % NT)),
            pl.BlockSpec((FF, C), lambda i: (0, 0)),
            pl.BlockSpec((FF, 1), lambda i: (0, 0)),
            pl.BlockSpec((C, FF), lambda i: (0, 0)),
            pl.BlockSpec((C, 1), lambda i: (0, 0)),
        ],
        out_specs=pl.BlockSpec((1, TT, C), lambda i: (i // NT, i % NT, 0)),
        out_shape=jax.ShapeDtypeStruct((B, S, C), jnp.float32),
        compiler_params=pltpu.CompilerParams(
            dimension_semantics=("arbitrary",)),
    )(x2, W1, b1.reshape(FF, 1), W2, b2.reshape(C, 1))


def kernel(x, Wg, We, be, W1, b1, W2, b2):
    xt = jnp.transpose(x, (0, 2, 1))
    gate, w0, w1, e0, e1, xtpk = _gating(xt, Wg)
    idxg, rw, bexp, nb, posA, posB = _routing_meta(
        e0.reshape(N), e1.reshape(N), w0.reshape(N), w1.reshape(N))
    xt_flat = xt.reshape(N, S)
    xs_pk = _sc_gather(xtpk.reshape(N, SP), idxg)
    partial = _expert_mm(xs_pk, We, be, rw, bexp, nb)
    x2t = _sc_combine(partial, xt_flat, posA, posB)
    return _ffn(x2t.reshape(B, C, S), W1, b1, W2, b2), gate


# speedup vs baseline: 1.0559x; 1.0559x over previous
import functools

import jax
import jax.numpy as jnp
from jax import lax
from jax.experimental import pallas as pl
from jax.experimental.pallas import tpu as pltpu
from jax.experimental.pallas import tpu_sc as plsc

B, S, C = 2, 2048, 768
E, K, FF = 8, 2, 2048
N = B * C
SP = S // 2
BLK = 128
NB = 32
NP = NB * BLK
TT = 512
NT = S // TT
FT = 512

_NC, _NS = 2, 16
_NW = _NC * _NS
GCH = 32
CCH = 8


def _pack_rows(x):
    u = lax.bitcast_convert_type(x, jnp.uint32)
    r = (u + jnp.uint32(0x7FFF) + ((u >> 16) & jnp.uint32(1))) >> 16
    k = x.shape[-1] // 2
    p = r[..., :k] | (r[..., k:] << 16)
    return lax.bitcast_convert_type(p, jnp.int32)


def _unpack_rows(p):
    u = lax.bitcast_convert_type(p, jnp.uint32)
    lo = lax.bitcast_convert_type(u << 16, jnp.float32)
    hi = lax.bitcast_convert_type(u & jnp.uint32(0xFFFF0000), jnp.float32)
    return jnp.concatenate([lo, hi], axis=-1)


def _gating_body(xt_ref, wg_ref, gate_ref, w0_ref, w1_ref, e0_ref, e1_ref,
                 xtpk_ref):
    xtb = xt_ref[0]
    xtpk_ref[0] = _pack_rows(xtb)
    gl = lax.dot_general(xtb, wg_ref[...], (((1,), (1,)), ((), ())),
                         preferred_element_type=jnp.float32)
    m = jnp.max(gl, axis=1, keepdims=True)
    ex = jnp.exp(gl - m)
    gate = ex / jnp.sum(ex, axis=1, keepdims=True)
    gate_ref[0] = gate
    iota = lax.broadcasted_iota(jnp.int32, (C, E), 1)
    m0 = jnp.max(gate, axis=1, keepdims=True)
    i0 = jnp.min(jnp.where(gate == m0, iota, E), axis=1, keepdims=True)
    g2 = jnp.where(iota == i0, -jnp.inf, gate)
    m1 = jnp.max(g2, axis=1, keepdims=True)
    i1 = jnp.min(jnp.where(g2 == m1, iota, E), axis=1, keepdims=True)
    w0_ref[0] = m0
    w1_ref[0] = m1
    e0_ref[0] = i0
    e1_ref[0] = i1


def _gating(xt, Wg):
    out_shapes = (
        jax.ShapeDtypeStruct((B, C, E), jnp.float32),
        jax.ShapeDtypeStruct((B, C, 1), jnp.float32),
        jax.ShapeDtypeStruct((B, C, 1), jnp.float32),
        jax.ShapeDtypeStruct((B, C, 1), jnp.int32),
        jax.ShapeDtypeStruct((B, C, 1), jnp.int32),
        jax.ShapeDtypeStruct((B, C, SP), jnp.int32),
    )
    return pl.pallas_call(
        _gating_body,
        grid=(B,),
        in_specs=[
            pl.BlockSpec((1, C, S), lambda b: (b, 0, 0)),
            pl.BlockSpec((E, S), lambda b: (0, 0)),
        ],
        out_specs=(
            pl.BlockSpec((1, C, E), lambda b: (b, 0, 0)),
            pl.BlockSpec((1, C, 1), lambda b: (b, 0, 0)),
            pl.BlockSpec((1, C, 1), lambda b: (b, 0, 0)),
            pl.BlockSpec((1, C, 1), lambda b: (b, 0, 0)),
            pl.BlockSpec((1, C, 1), lambda b: (b, 0, 0)),
            pl.BlockSpec((1, C, SP), lambda b: (b, 0, 0)),
        ),
        out_shape=out_shapes,
    )(xt, Wg)


def _routing_meta(e0, e1, w0, w1):
    eP = jnp.concatenate([e0, e1])
    wP = jnp.concatenate([w0, w1])
    tokP = jnp.concatenate([jnp.arange(N, dtype=jnp.int32)] * 2)
    oh = (eP[:, None] == jnp.arange(E, dtype=jnp.int32)[None, :]).astype(jnp.int32)
    csum = jnp.cumsum(oh, axis=0)
    cnt = csum[-1]
    rank = jnp.take_along_axis(csum, eP[:, None], axis=1)[:, 0] - 1
    blocks_e = (cnt + BLK - 1) // BLK
    bcum = jnp.cumsum(blocks_e)
    bstart = bcum - blocks_e
    pos = bstart[eP] * BLK + rank
    idxg = jnp.zeros((NP,), jnp.int32).at[pos].set(tokP)
    rw = jnp.zeros((NP,), jnp.float32).at[pos].set(wP)
    bexp = jnp.minimum(
        jnp.searchsorted(bcum, jnp.arange(NB, dtype=jnp.int32), side="right"),
        E - 1).astype(jnp.int32)
    nb = bcum[-1:].astype(jnp.int32)
    return idxg, rw, bexp, nb, pos[:N], pos[N:]


def _sc_gather(tab, idx_h):
    mesh = plsc.VectorSubcoreMesh(core_axis_name="c", subcore_axis_name="s")
    dt = tab.dtype
    D = tab.shape[1]
    nslots = idx_h.shape[0]
    rpw = nslots // _NW
    nch = rpw // GCH

    @functools.partial(
        pl.kernel,
        out_type=jax.ShapeDtypeStruct((nslots, D), dt),
        mesh=mesh,
        scratch_types=[
            pltpu.VMEM((rpw,), jnp.int32),
            pltpu.VMEM((GCH, D), dt),
            pltpu.VMEM((GCH, D), dt),
            pltpu.VMEM((GCH, D), dt),
        ] + [pltpu.SemaphoreType.DMA] * 6,
    )
    def k(tab_hbm, idx_hbm, out_hbm, idx_all, r0, r1, r2,
          sg0, sg1, sg2, ss0, ss1, ss2):
        wid = lax.axis_index("s") * _NC + lax.axis_index("c")
        base = wid * rpw
        rows = [r0, r1, r2]
        semg = [sg0, sg1, sg2]
        sems = [ss0, ss1, ss2]
        g = [None, None, None]
        s = [None, None, None]
        pltpu.sync_copy(idx_hbm.at[pl.ds(base, rpw)], idx_all)

        def gstart(ci, b):
            g[b] = pltpu.async_copy(
                tab_hbm.at[idx_all.at[pl.ds(ci * GCH, GCH)]], rows[b], semg[b])

        for b in range(min(3, nch)):
            gstart(b, b)
        for ci in range(nch):
            b = ci % 3
            g[b].wait()
            s[b] = pltpu.async_copy(
                rows[b], out_hbm.at[pl.ds(base + ci * GCH, GCH)], sems[b])
            if ci + 3 < nch:
                s[b].wait()
                gstart(ci + 3, b)
        for ci in range(max(0, nch - 3), nch):
            s[ci % 3].wait()

    return k(tab, idx_h)


def _mm_body(bexp_sm, nb_sm, xs_ref, we_ref, rw_ref, be_ref, out_ref):
    j = pl.program_id(0)

    @pl.when(j < nb_sm[0])
    def _():
        x = _unpack_rows(xs_ref[pl.ds(j * BLK, BLK), :])
        w = we_ref[0]
        acc = lax.dot_general(x, w, (((1,), (1,)), ((), ())),
                              preferred_element_type=jnp.float32)
        out_ref[...] = (acc + be_ref[0]) * rw_ref[...]


def _expert_mm(xs_pk, We, be, rw, bexp, nb):
    be3 = be.reshape(E, 1, S)
    rw2 = rw.reshape(NP, 1)
    grid_spec = pltpu.PrefetchScalarGridSpec(
        num_scalar_prefetch=2,
        grid=(NB,),
        in_specs=[
            pl.BlockSpec((NP, SP), lambda j, bexp, nb: (0, 0)),
            pl.BlockSpec((1, S, S), lambda j, bexp, nb: (bexp[j], 0, 0)),
            pl.BlockSpec((BLK, 1), lambda j, bexp, nb: (j, 0)),
            pl.BlockSpec((1, 1, S), lambda j, bexp, nb: (bexp[j], 0, 0)),
        ],
        out_specs=pl.BlockSpec((BLK, S), lambda j, bexp, nb: (j, 0)),
    )
    return pl.pallas_call(
        _mm_body,
        grid_spec=grid_spec,
        out_shape=jax.ShapeDtypeStruct((NP, S), jnp.float32),
        compiler_params=pltpu.CompilerParams(
            dimension_semantics=("arbitrary",)),
    )(bexp, nb, xs_pk, We, rw2, be3)


def _sc_combine(partial, xt_sub, posA, posB):
    mesh = plsc.VectorSubcoreMesh(core_axis_name="c", subcore_axis_name="s")
    ntok = posA.shape[0]
    tpw = ntok // _NW
    nchc = tpw // CCH

    @functools.partial(
        pl.kernel,
        out_type=jax.ShapeDtypeStruct((ntok, S), jnp.float32),
        mesh=mesh,
        scratch_types=[
            pltpu.VMEM((tpw,), jnp.int32),
            pltpu.VMEM((tpw,), jnp.int32),
            pltpu.VMEM((CCH, S), jnp.float32),
            pltpu.VMEM((CCH, S), jnp.float32),
            pltpu.VMEM((CCH, S), jnp.float32),
            pltpu.VMEM((CCH, S), jnp.float32),
            pltpu.VMEM((CCH, S), jnp.float32),
            pltpu.VMEM((CCH, S), jnp.float32),
        ] + [pltpu.SemaphoreType.DMA] * 8,
    )
    def k(p_hbm, xt_hbm, pa_hbm, pb_hbm, out_hbm,
          ia_all, ib_all, pva0, pva1, pvb0, pvb1, xtv0, xtv1,
          spa0, spa1, spb0, spb1, sx0, sx1, so0, so1):
        wid = lax.axis_index("s") * _NC + lax.axis_index("c")
        base = wid * tpw
        pva = [pva0, pva1]
        pvb = [pvb0, pvb1]
        xtv = [xtv0, xtv1]
        spa = [spa0, spa1]
        spb = [spb0, spb1]
        sx = [sx0, sx1]
        sso = [so0, so1]
        ga = [None, None]
        gb = [None, None]
        gx = [None, None]
        so = [None, None]
        pltpu.sync_copy(pa_hbm.at[pl.ds(base, tpw)], ia_all)
        pltpu.sync_copy(pb_hbm.at[pl.ds(base, tpw)], ib_all)

        def start(ci, b):
            off = base + ci * CCH
            ga[b] = pltpu.async_copy(
                p_hbm.at[ia_all.at[pl.ds(ci * CCH, CCH)]], pva[b], spa[b])
            gb[b] = pltpu.async_copy(
                p_hbm.at[ib_all.at[pl.ds(ci * CCH, CCH)]], pvb[b], spb[b])
            gx[b] = pltpu.async_copy(
                xt_hbm.at[pl.ds(off, CCH)], xtv[b], sx[b])

        start(0, 0)
        for ci in range(nchc):
            b = ci % 2
            if ci + 1 < nchc:
                nb_ = (ci + 1) % 2
                if so[nb_] is not None:
                    so[nb_].wait()
                    so[nb_] = None
                start(ci + 1, nb_)
            ga[b].wait()
            gb[b].wait()
            gx[b].wait()
            for ti in range(CCH):
                def colbody(cc, carry2, ti=ti, b=b):
                    cs = cc * 16
                    va = pva[b][ti, pl.ds(cs, 16)]
                    vb = pvb[b][ti, pl.ds(cs, 16)]
                    xv = xtv[b][ti, pl.ds(cs, 16)]
                    xtv[b][ti, pl.ds(cs, 16)] = jnp.maximum(va + vb, 0.0) + xv
                    return carry2
                lax.fori_loop(0, S // 16, colbody, 0, unroll=4)
            so[b] = pltpu.async_copy(
                xtv[b], out_hbm.at[pl.ds(base + ci * CCH, CCH)], sso[b])
        for b in range(2):
            if so[b] is not None:
                so[b].wait()

    return k(partial, xt_sub, posA, posB)


def _ffn_body(x2_ref, w1_ref, b1_ref, w2_ref, b2_ref, out_ref):
    xb = x2_ref[0]
    acc = jnp.zeros((C, TT), jnp.float32)
    for fi in range(FF // FT):
        h = lax.dot_general(w1_ref[pl.ds(fi * FT, FT), :], xb,
                            (((1,), (0,)), ((), ())),
                            preferred_element_type=jnp.float32)
        h = jnp.maximum(h + b1_ref[pl.ds(fi * FT, FT), :], 0.0)
        acc = acc + lax.dot_general(w2_ref[:, pl.ds(fi * FT, FT)], h,
                                    (((1,), (0,)), ((), ())),
                                    preferred_element_type=jnp.float32)
    out_ref[0] = jnp.transpose(acc + b2_ref[...] + xb)


def _ffn(x2, W1, b1, W2, b2):
    return pl.pallas_call(
        _ffn_body,
        grid=(B * NT,),
        in_specs=[
            pl.BlockSpec((1, C, TT), lambda i: (i // NT, 0, i % NT)),
            pl.BlockSpec((FF, C), lambda i: (0, 0)),
            pl.BlockSpec((FF, 1), lambda i: (0, 0)),
            pl.BlockSpec((C, FF), lambda i: (0, 0)),
            pl.BlockSpec((C, 1), lambda i: (0, 0)),
        ],
        out_specs=pl.BlockSpec((1, TT, C), lambda i: (i // NT, i % NT, 0)),
        out_shape=jax.ShapeDtypeStruct((B, S, C), jnp.float32),
        compiler_params=pltpu.CompilerParams(
            dimension_semantics=("arbitrary",)),
    )(x2, W1, b1.reshape(FF, 1), W2, b2.reshape(C, 1))


def kernel(x, Wg, We, be, W1, b1, W2, b2):
    xt = jnp.transpose(x, (0, 2, 1))
    gate, w0, w1, e0, e1, xtpk = _gating(xt, Wg)
    idxg, rw, bexp, nb, posA, posB = _routing_meta(
        e0.reshape(N), e1.reshape(N), w0.reshape(N), w1.reshape(N))
    xt_flat = xt.reshape(N, S)
    xs_pk = _sc_gather(xtpk.reshape(N, SP), idxg)
    partial = _expert_mm(xs_pk, We, be, rw, bexp, nb)
    x2t = _sc_combine(partial, xt_flat, posA, posB)
    return _ffn(x2t.reshape(B, C, S), W1, b1, W2, b2), gate


# speedup vs baseline: 1.0932x; 1.0353x over previous
import functools

import jax
import jax.numpy as jnp
from jax import lax
from jax.experimental import pallas as pl
from jax.experimental.pallas import tpu as pltpu
from jax.experimental.pallas import tpu_sc as plsc

B, S, C = 2, 2048, 768
E, K, FF = 8, 2, 2048
N = B * C
SP = S // 2
BLK = 128
NB = 32
NP = NB * BLK
TT = 512
NT = S // TT
FT = 512

_NC, _NS = 2, 16
_NW = _NC * _NS
GCH = 16
GNB = 6
CCH = 8


def _pack_rows(x):
    u = lax.bitcast_convert_type(x, jnp.uint32)
    r = (u + jnp.uint32(0x7FFF) + ((u >> 16) & jnp.uint32(1))) >> 16
    k = x.shape[-1] // 2
    p = r[..., :k] | (r[..., k:] << 16)
    return lax.bitcast_convert_type(p, jnp.int32)


def _unpack_rows(p):
    u = lax.bitcast_convert_type(p, jnp.uint32)
    lo = lax.bitcast_convert_type(u << 16, jnp.float32)
    hi = lax.bitcast_convert_type(u & jnp.uint32(0xFFFF0000), jnp.float32)
    return jnp.concatenate([lo, hi], axis=-1)


def _gating_body(x_ref, wg_ref, gate_ref, w0_ref, w1_ref, e0_ref, e1_ref,
                 xtpk_ref, xt_ref):
    xb = x_ref[0]
    xtb = jnp.transpose(xb)
    xt_ref[0] = xtb
    xtpk_ref[0] = _pack_rows(xtb)
    gl = lax.dot_general(xb, wg_ref[...], (((0,), (1,)), ((), ())),
                         preferred_element_type=jnp.float32)
    m = jnp.max(gl, axis=1, keepdims=True)
    ex = jnp.exp(gl - m)
    gate = ex / jnp.sum(ex, axis=1, keepdims=True)
    gate_ref[0] = gate
    iota = lax.broadcasted_iota(jnp.int32, (C, E), 1)
    m0 = jnp.max(gate, axis=1, keepdims=True)
    i0 = jnp.min(jnp.where(gate == m0, iota, E), axis=1, keepdims=True)
    g2 = jnp.where(iota == i0, -jnp.inf, gate)
    m1 = jnp.max(g2, axis=1, keepdims=True)
    i1 = jnp.min(jnp.where(g2 == m1, iota, E), axis=1, keepdims=True)
    w0_ref[0] = m0
    w1_ref[0] = m1
    e0_ref[0] = i0
    e1_ref[0] = i1


def _gating(x, Wg):
    out_shapes = (
        jax.ShapeDtypeStruct((B, C, E), jnp.float32),
        jax.ShapeDtypeStruct((B, C, 1), jnp.float32),
        jax.ShapeDtypeStruct((B, C, 1), jnp.float32),
        jax.ShapeDtypeStruct((B, C, 1), jnp.int32),
        jax.ShapeDtypeStruct((B, C, 1), jnp.int32),
        jax.ShapeDtypeStruct((B, C, SP), jnp.int32),
        jax.ShapeDtypeStruct((B, C, S), jnp.float32),
    )
    return pl.pallas_call(
        _gating_body,
        grid=(B,),
        in_specs=[
            pl.BlockSpec((1, S, C), lambda b: (b, 0, 0)),
            pl.BlockSpec((E, S), lambda b: (0, 0)),
        ],
        out_specs=(
            pl.BlockSpec((1, C, E), lambda b: (b, 0, 0)),
            pl.BlockSpec((1, C, 1), lambda b: (b, 0, 0)),
            pl.BlockSpec((1, C, 1), lambda b: (b, 0, 0)),
            pl.BlockSpec((1, C, 1), lambda b: (b, 0, 0)),
            pl.BlockSpec((1, C, 1), lambda b: (b, 0, 0)),
            pl.BlockSpec((1, C, SP), lambda b: (b, 0, 0)),
            pl.BlockSpec((1, C, S), lambda b: (b, 0, 0)),
        ),
        out_shape=out_shapes,
    )(x, Wg)


def _routing_meta(e0, e1, w0, w1):
    eP = jnp.concatenate([e0, e1])
    wP = jnp.concatenate([w0, w1])
    tokP = jnp.concatenate([jnp.arange(N, dtype=jnp.int32)] * 2)
    oh = (eP[:, None] == jnp.arange(E, dtype=jnp.int32)[None, :]).astype(jnp.int32)
    csum = jnp.cumsum(oh, axis=0)
    cnt = csum[-1]
    rank = jnp.take_along_axis(csum, eP[:, None], axis=1)[:, 0] - 1
    blocks_e = (cnt + BLK - 1) // BLK
    bcum = jnp.cumsum(blocks_e)
    bstart = bcum - blocks_e
    pos = bstart[eP] * BLK + rank
    idxg = jnp.zeros((NP,), jnp.int32).at[pos].set(tokP)
    rw = jnp.zeros((NP,), jnp.float32).at[pos].set(wP)
    bexp = jnp.minimum(
        jnp.searchsorted(bcum, jnp.arange(NB, dtype=jnp.int32), side="right"),
        E - 1).astype(jnp.int32)
    nb = bcum[-1:].astype(jnp.int32)
    return idxg, rw, bexp, nb, pos[:N], pos[N:]


def _sc_gather(tab, idx_h):
    mesh = plsc.VectorSubcoreMesh(core_axis_name="c", subcore_axis_name="s")
    dt = tab.dtype
    D = tab.shape[1]
    nslots = idx_h.shape[0]
    rpw = nslots // _NW
    nch = rpw // GCH

    @functools.partial(
        pl.kernel,
        out_type=jax.ShapeDtypeStruct((nslots, D), dt),
        mesh=mesh,
        scratch_types=(
            [pltpu.VMEM((rpw,), jnp.int32)]
            + [pltpu.VMEM((GCH, D), dt)] * GNB
            + [pltpu.SemaphoreType.DMA] * (2 * GNB)
        ),
    )
    def k(tab_hbm, idx_hbm, out_hbm, idx_all, *bufs_sems):
        wid = lax.axis_index("s") * _NC + lax.axis_index("c")
        base = wid * rpw
        rows = list(bufs_sems[:GNB])
        semg = list(bufs_sems[GNB:2 * GNB])
        sems = list(bufs_sems[2 * GNB:])
        g = [None] * GNB
        s = [None] * GNB
        pltpu.sync_copy(idx_hbm.at[pl.ds(base, rpw)], idx_all)

        def gstart(ci, b):
            g[b] = pltpu.async_copy(
                tab_hbm.at[idx_all.at[pl.ds(ci * GCH, GCH)]], rows[b], semg[b])

        for b in range(min(GNB, nch)):
            gstart(b, b)
        for ci in range(nch):
            b = ci % GNB
            g[b].wait()
            s[b] = pltpu.async_copy(
                rows[b], out_hbm.at[pl.ds(base + ci * GCH, GCH)], sems[b])
            if ci + GNB < nch:
                s[b].wait()
                gstart(ci + GNB, b)
        for ci in range(max(0, nch - GNB), nch):
            s[ci % GNB].wait()

    return k(tab, idx_h)


def _mm_body(bexp_sm, nb_sm, xs_ref, we_ref, rw_ref, be_ref, out_ref):
    j = pl.program_id(0)

    @pl.when(j < nb_sm[0])
    def _():
        x = _unpack_rows(xs_ref[pl.ds(j * BLK, BLK), :])
        w = we_ref[0]
        acc = lax.dot_general(x, w, (((1,), (1,)), ((), ())),
                              preferred_element_type=jnp.float32)
        out_ref[...] = (acc + be_ref[0]) * rw_ref[...]


def _expert_mm(xs_pk, We, be, rw, bexp, nb):
    be3 = be.reshape(E, 1, S)
    rw2 = rw.reshape(NP, 1)
    grid_spec = pltpu.PrefetchScalarGridSpec(
        num_scalar_prefetch=2,
        grid=(NB,),
        in_specs=[
            pl.BlockSpec((NP, SP), lambda j, bexp, nb: (0, 0)),
            pl.BlockSpec((1, S, S), lambda j, bexp, nb: (bexp[j], 0, 0)),
            pl.BlockSpec((BLK, 1), lambda j, bexp, nb: (j, 0)),
            pl.BlockSpec((1, 1, S), lambda j, bexp, nb: (bexp[j], 0, 0)),
        ],
        out_specs=pl.BlockSpec((BLK, S), lambda j, bexp, nb: (j, 0)),
    )
    return pl.pallas_call(
        _mm_body,
        grid_spec=grid_spec,
        out_shape=jax.ShapeDtypeStruct((NP, S), jnp.float32),
        compiler_params=pltpu.CompilerParams(
            dimension_semantics=("arbitrary",)),
    )(bexp, nb, xs_pk, We, rw2, be3)


def _sc_combine(partial, xt_sub, posA, posB):
    mesh = plsc.VectorSubcoreMesh(core_axis_name="c", subcore_axis_name="s")
    ntok = posA.shape[0]
    tpw = ntok // _NW
    nchc = tpw // CCH

    @functools.partial(
        pl.kernel,
        out_type=jax.ShapeDtypeStruct((ntok, S), jnp.float32),
        mesh=mesh,
        scratch_types=[
            pltpu.VMEM((tpw,), jnp.int32),
            pltpu.VMEM((tpw,), jnp.int32),
            pltpu.VMEM((CCH, S), jnp.float32),
            pltpu.VMEM((CCH, S), jnp.float32),
            pltpu.VMEM((CCH, S), jnp.float32),
            pltpu.VMEM((CCH, S), jnp.float32),
            pltpu.VMEM((CCH, S), jnp.float32),
            pltpu.VMEM((CCH, S), jnp.float32),
        ] + [pltpu.SemaphoreType.DMA] * 8,
    )
    def k(p_hbm, xt_hbm, pa_hbm, pb_hbm, out_hbm,
          ia_all, ib_all, pva0, pva1, pvb0, pvb1, xtv0, xtv1,
          spa0, spa1, spb0, spb1, sx0, sx1, so0, so1):
        wid = lax.axis_index("s") * _NC + lax.axis_index("c")
        base = wid * tpw
        pva = [pva0, pva1]
        pvb = [pvb0, pvb1]
        xtv = [xtv0, xtv1]
        spa = [spa0, spa1]
        spb = [spb0, spb1]
        sx = [sx0, sx1]
        sso = [so0, so1]
        ga = [None, None]
        gb = [None, None]
        gx = [None, None]
        so = [None, None]
        pltpu.sync_copy(pa_hbm.at[pl.ds(base, tpw)], ia_all)
        pltpu.sync_copy(pb_hbm.at[pl.ds(base, tpw)], ib_all)

        def start(ci, b):
            off = base + ci * CCH
            ga[b] = pltpu.async_copy(
                p_hbm.at[ia_all.at[pl.ds(ci * CCH, CCH)]], pva[b], spa[b])
            gb[b] = pltpu.async_copy(
                p_hbm.at[ib_all.at[pl.ds(ci * CCH, CCH)]], pvb[b], spb[b])
            gx[b] = pltpu.async_copy(
                xt_hbm.at[pl.ds(off, CCH)], xtv[b], sx[b])

        start(0, 0)
        for ci in range(nchc):
            b = ci % 2
            if ci + 1 < nchc:
                nb_ = (ci + 1) % 2
                if so[nb_] is not None:
                    so[nb_].wait()
                    so[nb_] = None
                start(ci + 1, nb_)
            ga[b].wait()
            gb[b].wait()
            gx[b].wait()
            for ti in range(CCH):
                def colbody(cc, carry2, ti=ti, b=b):
                    cs = cc * 16
                    va = pva[b][ti, pl.ds(cs, 16)]
                    vb = pvb[b][ti, pl.ds(cs, 16)]
                    xv = xtv[b][ti, pl.ds(cs, 16)]
                    xtv[b][ti, pl.ds(cs, 16)] = jnp.maximum(va + vb, 0.0) + xv
                    return carry2
                lax.fori_loop(0, S // 16, colbody, 0, unroll=4)
            so[b] = pltpu.async_copy(
                xtv[b], out_hbm.at[pl.ds(base + ci * CCH, CCH)], sso[b])
        for b in range(2):
            if so[b] is not None:
                so[b].wait()

    return k(partial, xt_sub, posA, posB)


def _ffn_body(x2_ref, w1_ref, b1_ref, w2_ref, b2_ref, out_ref):
    xb = x2_ref[0]
    acc = jnp.zeros((C, TT), jnp.float32)
    for fi in range(FF // FT):
        h = lax.dot_general(w1_ref[pl.ds(fi * FT, FT), :], xb,
                            (((1,), (0,)), ((), ())),
                            preferred_element_type=jnp.float32)
        h = jnp.maximum(h + b1_ref[pl.ds(fi * FT, FT), :], 0.0)
        acc = acc + lax.dot_general(w2_ref[:, pl.ds(fi * FT, FT)], h,
                                    (((1,), (0,)), ((), ())),
                                    preferred_element_type=jnp.float32)
    out_ref[0] = jnp.transpose(acc + b2_ref[...] + xb)


def _ffn(x2, W1, b1, W2, b2):
    return pl.pallas_call(
        _ffn_body,
        grid=(B * NT,),
        in_specs=[
            pl.BlockSpec((1, C, TT), lambda i: (i // NT, 0, i % NT)),
            pl.BlockSpec((FF, C), lambda i: (0, 0)),
            pl.BlockSpec((FF, 1), lambda i: (0, 0)),
            pl.BlockSpec((C, FF), lambda i: (0, 0)),
            pl.BlockSpec((C, 1), lambda i: (0, 0)),
        ],
        out_specs=pl.BlockSpec((1, TT, C), lambda i: (i // NT, i % NT, 0)),
        out_shape=jax.ShapeDtypeStruct((B, S, C), jnp.float32),
        compiler_params=pltpu.CompilerParams(
            dimension_semantics=("arbitrary",)),
    )(x2, W1, b1.reshape(FF, 1), W2, b2.reshape(C, 1))


def kernel(x, Wg, We, be, W1, b1, W2, b2):
    gate, w0, w1, e0, e1, xtpk, xt = _gating(x, Wg)
    idxg, rw, bexp, nb, posA, posB = _routing_meta(
        e0.reshape(N), e1.reshape(N), w0.reshape(N), w1.reshape(N))
    xs_pk = _sc_gather(xtpk.reshape(N, SP), idxg)
    partial = _expert_mm(xs_pk, We, be, rw, bexp, nb)
    x2t = _sc_combine(partial, xt.reshape(N, S), posA, posB)
    return _ffn(x2t.reshape(B, C, S), W1, b1, W2, b2), gate
